# Initial kernel scaffold; baseline (speedup 1.0000x reference)
import jax
import jax.numpy as jnp
from jax.experimental import pallas as pl


def kernel(x_lnc_jaccard, x_prot_jaccard, x_lnc_blast, x_prot_blast, edge_index_lnc_jaccard, edge_index_prot_jaccard, edge_index_lnc_blast, edge_index_prot_blast, W_j1_lnc, b_j1_lnc, W_j1_prot, b_j1_prot, W_j2_lnc, b_j2_lnc, W_j2_prot, b_j2_prot, W_b1_lnc, b_b1_lnc, W_b1_prot, b_b1_prot, W_b2_lnc, b_b2_lnc, W_b2_prot, b_b2_prot, W_res_lnc, b_res_lnc, W_res_prot, b_res_prot):
    raise NotImplementedError("write your pallas kernel here")


# speedup vs baseline: 19.3275x; 19.3275x over previous
import functools

import jax
import jax.numpy as jnp
from jax import lax
from jax.experimental import pallas as pl
from jax.experimental.pallas import tpu as pltpu
import jax.experimental.pallas.tpu_sc as plsc

N = 10000
E = 320000
D_IN, D_HID, D_OUT = 128, 128, 64
NC, NS = 2, 16
CH = 128
BLK = 8
N_BLOCKS = 20
EPT = N_BLOCKS * BLK * CH
E_PAD = EPT * NS
RPT_IDX = EPT // CH
N_ACC = N + 112
RPT = N_ACC // NS
N_DEG = 10240
DPT = N_DEG // NS
BM = 2000


def _mesh():
    return plsc.VectorSubcoreMesh(core_axis_name="c", subcore_axis_name="s",
                                  num_cores=NC, num_subcores=NS)


@functools.cache
def _deg_kernel():
    @functools.partial(
        pl.kernel, mesh=_mesh(),
        out_type=jax.ShapeDtypeStruct((4 * N_DEG,), jnp.float32),
        scratch_types=[
            pltpu.VMEM((BLK, CH), jnp.int32),
            pltpu.VMEM((DPT,), jnp.float32),
            pltpu.VMEM_SHARED((N_DEG,), jnp.float32),
        ],
        name="gcn_deg",
    )
    def k(didx, out, didx_v, ones_v, deg):
        c = lax.axis_index("c")
        s = lax.axis_index("s")
        for i in range(DPT // 16):
            ones_v[pl.ds(i * 16, 16)] = jnp.ones((16,), jnp.float32)
        for p in range(2):
            pi = 2 * c + p
            pltpu.sync_copy(ones_v, deg.at[pl.ds(s * DPT, DPT)])
            plsc.subcore_barrier()
            idx_base = pi * (NS * RPT_IDX) + s * RPT_IDX

            @pl.loop(0, N_BLOCKS)
            def _(b):
                pltpu.sync_copy(didx.at[pl.ds(idx_base + b * BLK, BLK), :],
                                didx_v)
                for j in range(BLK):
                    pltpu.sync_copy(ones_v.at[pl.ds(0, CH)],
                                    deg.at[didx_v.at[j]], add=True)

            plsc.subcore_barrier()
            pltpu.sync_copy(deg.at[pl.ds(s * DPT, DPT)],
                            out.at[pl.ds(pi * N_DEG + s * DPT, DPT)])
            plsc.subcore_barrier()

    return k


@functools.cache
def _agg_kernel(d):
    @functools.partial(
        pl.kernel, mesh=_mesh(),
        out_type=jax.ShapeDtypeStruct((4 * N_ACC, d), jnp.float32),
        scratch_types=[
            pltpu.VMEM((BLK, CH), jnp.int32),
            pltpu.VMEM((BLK, CH), jnp.int32),
            pltpu.VMEM((CH, d), jnp.float32),
            pltpu.VMEM_SHARED((N_ACC, d), jnp.float32),
            pltpu.SemaphoreType.DMA,
        ],
        compiler_params=pltpu.CompilerParams(
            use_tc_tiling_on_sc=(d == 128)),
        name=f"gcn_agg_d{d}",
    )
    def k(tables, sidx, didx, out, sidx_v, didx_v, rows_v, acc, sem):
        c = lax.axis_index("c")
        s = lax.axis_index("s")
        for p in range(2):
            pi = 2 * c + p
            pltpu.sync_copy(tables.at[pl.ds(pi * N_ACC + s * RPT, RPT), :],
                            acc.at[pl.ds(s * RPT, RPT), :])
            plsc.subcore_barrier()
            idx_base = pi * (NS * RPT_IDX) + s * RPT_IDX

            @pl.loop(0, N_BLOCKS)
            def _(b):
                r0 = idx_base + b * BLK
                pltpu.sync_copy(sidx.at[pl.ds(r0, BLK), :], sidx_v)
                pltpu.sync_copy(didx.at[pl.ds(r0, BLK), :], didx_v)
                for j in range(BLK):
                    pltpu.async_copy(tables.at[sidx_v.at[j]], rows_v,
                                     sem).wait()
                    pltpu.sync_copy(rows_v, acc.at[didx_v.at[j]], add=True)

            plsc.subcore_barrier()
            pltpu.sync_copy(acc.at[pl.ds(s * RPT, RPT), :],
                            out.at[pl.ds(pi * N_ACC + s * RPT, RPT), :])
            plsc.subcore_barrier()

    return k


def _lrelu(v):
    return jnp.where(v >= 0, v, 0.2 * v)


def _tc1(deg4, xs4, w1s):
    def body(deg_ref, x_ref, w_ref, hp_ref, dv_ref):
        dv = 1.0 / jnp.sqrt(deg_ref[0])
        h = jnp.dot(x_ref[0], w_ref[0], preferred_element_type=jnp.float32)
        hp_ref[0] = h * dv
        dv_ref[0] = dv

    return pl.pallas_call(
        body,
        grid=(4, N // BM),
        in_specs=[pl.BlockSpec((1, BM, 1), lambda i, j: (i, j, 0)),
                  pl.BlockSpec((1, BM, D_IN), lambda i, j: (i, j, 0)),
                  pl.BlockSpec((1, D_IN, D_HID), lambda i, j: (i, 0, 0))],
        out_specs=[pl.BlockSpec((1, BM, D_HID), lambda i, j: (i, j, 0)),
                   pl.BlockSpec((1, BM, 1), lambda i, j: (i, j, 0))],
        out_shape=[jax.ShapeDtypeStruct((4, N, D_HID), jnp.float32),
                   jax.ShapeDtypeStruct((4, N, 1), jnp.float32)],
    )(deg4, xs4, w1s)


def _tc2(acc1, dinv, b1s, w2s):
    def body(a_ref, dv_ref, b_ref, w_ref, o_ref):
        dv = dv_ref[0]
        z = _lrelu(a_ref[0] * dv + b_ref[0])
        o_ref[0] = jnp.dot(z, w_ref[0],
                           preferred_element_type=jnp.float32) * dv

    return pl.pallas_call(
        body,
        grid=(4, N // BM),
        in_specs=[pl.BlockSpec((1, BM, D_HID), lambda i, j: (i, j, 0)),
                  pl.BlockSpec((1, BM, 1), lambda i, j: (i, j, 0)),
                  pl.BlockSpec((1, 1, D_HID), lambda i, j: (i, 0, 0)),
                  pl.BlockSpec((1, D_HID, D_OUT), lambda i, j: (i, 0, 0))],
        out_specs=pl.BlockSpec((1, BM, D_OUT), lambda i, j: (i, j, 0)),
        out_shape=jax.ShapeDtypeStruct((4, N, D_OUT), jnp.float32),
    )(acc1, dinv, b1s, w2s)


def _tc3(acc2, dinv, b2s):
    def body(a_ref, dv_ref, b_ref, o_ref):
        o_ref[0] = _lrelu(a_ref[0] * dv_ref[0] + b_ref[0])

    return pl.pallas_call(
        body,
        grid=(4, N // BM),
        in_specs=[pl.BlockSpec((1, BM, D_OUT), lambda i, j: (i, j, 0)),
                  pl.BlockSpec((1, BM, 1), lambda i, j: (i, j, 0)),
                  pl.BlockSpec((1, 1, D_OUT), lambda i, j: (i, 0, 0))],
        out_specs=pl.BlockSpec((1, BM, D_OUT), lambda i, j: (i, j, 0)),
        out_shape=jax.ShapeDtypeStruct((4, N, D_OUT), jnp.float32),
    )(acc2, dinv, b2s)


def _tc4(o4, xj2, wres, bres):
    def body(o1_ref, o2_ref, x_ref, w_ref, b_ref, c_ref):
        r = jnp.dot(x_ref[0], w_ref[0], preferred_element_type=jnp.float32)
        c_ref[0] = (o1_ref[0] + o2_ref[0]) * 0.5 + r + b_ref[0]

    return pl.pallas_call(
        body,
        grid=(2, N // BM),
        in_specs=[pl.BlockSpec((1, BM, D_OUT), lambda i, j: (i, j, 0)),
                  pl.BlockSpec((1, BM, D_OUT), lambda i, j: (i + 2, j, 0)),
                  pl.BlockSpec((1, BM, D_IN), lambda i, j: (i, j, 0)),
                  pl.BlockSpec((1, D_IN, D_OUT), lambda i, j: (i, 0, 0)),
                  pl.BlockSpec((1, 1, D_OUT), lambda i, j: (i, 0, 0))],
        out_specs=pl.BlockSpec((1, BM, D_OUT), lambda i, j: (i, j, 0)),
        out_shape=jax.ShapeDtypeStruct((2, N, D_OUT), jnp.float32),
    )(o4, o4, xj2, wres, bres)


def kernel(x_lnc_jaccard, x_prot_jaccard, x_lnc_blast, x_prot_blast,
           edge_index_lnc_jaccard, edge_index_prot_jaccard,
           edge_index_lnc_blast, edge_index_prot_blast,
           W_j1_lnc, b_j1_lnc, W_j1_prot, b_j1_prot,
           W_j2_lnc, b_j2_lnc, W_j2_prot, b_j2_prot,
           W_b1_lnc, b_b1_lnc, W_b1_prot, b_b1_prot,
           W_b2_lnc, b_b2_lnc, W_b2_prot, b_b2_prot,
           W_res_lnc, b_res_lnc, W_res_prot, b_res_prot):
    eis = (edge_index_lnc_jaccard, edge_index_prot_jaccard,
           edge_index_lnc_blast, edge_index_prot_blast)
    pad = jnp.arange(E_PAD - E, dtype=jnp.int32) % 16
    sidx_l, didx_l = [], []
    for p, ei in enumerate(eis):
        s_ = jnp.concatenate([ei[0], pad]) + p * N_ACC
        d_ = jnp.concatenate([ei[1], N + pad])
        sidx_l.append(s_.reshape(NS * RPT_IDX, CH))
        didx_l.append(d_.reshape(NS * RPT_IDX, CH))
    sidx = jnp.concatenate(sidx_l, axis=0)
    didx = jnp.concatenate(didx_l, axis=0)

    deg4 = _deg_kernel()(didx).reshape(4, N_DEG, 1)[:, :N]

    xs4 = jnp.stack([x_lnc_jaccard, x_prot_jaccard, x_lnc_blast, x_prot_blast])
    w1s = jnp.stack([W_j1_lnc, W_j1_prot, W_b1_lnc, W_b1_prot])
    h1p, dinv = _tc1(deg4, xs4, w1s)

    t1 = jnp.pad(h1p, ((0, 0), (0, N_ACC - N), (0, 0)))
    acc1 = _agg_kernel(D_HID)(t1.reshape(4 * N_ACC, D_HID), sidx, didx)
    acc1 = acc1.reshape(4, N_ACC, D_HID)[:, :N]

    b1s = jnp.stack([b_j1_lnc, b_j1_prot, b_b1_lnc, b_b1_prot])[:, None, :]
    w2s = jnp.stack([W_j2_lnc, W_j2_prot, W_b2_lnc, W_b2_prot])
    h2p = _tc2(acc1, dinv, b1s, w2s)

    t2 = jnp.pad(h2p, ((0, 0), (0, N_ACC - N), (0, 0)))
    acc2 = _agg_kernel(D_OUT)(t2.reshape(4 * N_ACC, D_OUT), sidx, didx)
    acc2 = acc2.reshape(4, N_ACC, D_OUT)[:, :N]

    b2s = jnp.stack([b_j2_lnc, b_j2_prot, b_b2_lnc, b_b2_prot])[:, None, :]
    o4 = _tc3(acc2, dinv, b2s)

    xj2 = jnp.stack([x_lnc_jaccard, x_prot_jaccard])
    wres = jnp.stack([W_res_lnc, W_res_prot])
    bres = jnp.stack([b_res_lnc, b_res_prot])[:, None, :]
    comb = _tc4(o4, xj2, wres, bres)

    return (comb[0], comb[1], o4[0], o4[1], o4[2], o4[3])


# speedup vs baseline: 26.5300x; 1.3727x over previous
import functools

import jax
import jax.numpy as jnp
from jax import lax
from jax.experimental import pallas as pl
from jax.experimental.pallas import tpu as pltpu
import jax.experimental.pallas.tpu_sc as plsc

N = 10000
E = 320000
D_IN, D_HID, D_OUT = 128, 128, 64
NC, NS = 2, 16
CH = 128
BLK = 8
N_BLOCKS = 20
EPT = N_BLOCKS * BLK * CH
E_PAD = EPT * NS
RPT_IDX = EPT // CH
N_ACC = N + 112
RPT = N_ACC // NS
N_DEG = 10240
DPT = N_DEG // NS
BM = 2000


def _mesh():
    return plsc.VectorSubcoreMesh(core_axis_name="c", subcore_axis_name="s",
                                  num_cores=NC, num_subcores=NS)


@functools.cache
def _deg_kernel():
    @functools.partial(
        pl.kernel, mesh=_mesh(),
        out_type=jax.ShapeDtypeStruct((4 * N_DEG,), jnp.float32),
        scratch_types=[
            pltpu.VMEM((BLK, CH), jnp.int32),
            pltpu.VMEM((DPT,), jnp.float32),
            pltpu.VMEM_SHARED((N_DEG,), jnp.float32),
        ],
        name="gcn_deg",
    )
    def k(didx, out, didx_v, ones_v, deg):
        c = lax.axis_index("c")
        s = lax.axis_index("s")
        for i in range(DPT // 16):
            ones_v[pl.ds(i * 16, 16)] = jnp.ones((16,), jnp.float32)
        for p in range(2):
            pi = 2 * c + p
            pltpu.sync_copy(ones_v, deg.at[pl.ds(s * DPT, DPT)])
            plsc.subcore_barrier()
            idx_base = pi * (NS * RPT_IDX) + s * RPT_IDX

            @pl.loop(0, N_BLOCKS)
            def _(b):
                pltpu.sync_copy(didx.at[pl.ds(idx_base + b * BLK, BLK), :],
                                didx_v)
                for j in range(BLK):
                    pltpu.sync_copy(ones_v.at[pl.ds(0, CH)],
                                    deg.at[didx_v.at[j]], add=True)

            plsc.subcore_barrier()
            pltpu.sync_copy(deg.at[pl.ds(s * DPT, DPT)],
                            out.at[pl.ds(pi * N_DEG + s * DPT, DPT)])
            plsc.subcore_barrier()

    return k


NBUF = 4
N_GRP = RPT_IDX // NBUF
DW = 64


@functools.cache
def _agg_kernel(ppc):
    nprod = NC * ppc

    @functools.partial(
        pl.kernel, mesh=_mesh(),
        out_type=jax.ShapeDtypeStruct((nprod * N_ACC, DW), jnp.float32),
        scratch_types=[
            pltpu.VMEM((RPT_IDX, CH), jnp.int32),
            pltpu.VMEM((RPT_IDX, CH), jnp.int32),
            pltpu.VMEM((NBUF, CH, DW), jnp.float32),
            pltpu.VMEM_SHARED((N_ACC, DW), jnp.float32),
            pltpu.SemaphoreType.DMA,
            pltpu.SemaphoreType.DMA,
            pltpu.SemaphoreType.DMA,
            pltpu.SemaphoreType.DMA,
        ],
        compiler_params=pltpu.CompilerParams(use_tc_tiling_on_sc=False),
        name=f"gcn_agg_p{ppc}",
    )
    def k(tables, sidx, didx, out, sidx_v, didx_v, rows_v, acc,
          sem0, sem1, sem2, sem3):
        sems = (sem0, sem1, sem2, sem3)
        c = lax.axis_index("c")
        s = lax.axis_index("s")

        @pl.loop(0, ppc)
        def _prod(p):
            pi = ppc * c + p
            pltpu.sync_copy(tables.at[pl.ds(pi * N_ACC + s * RPT, RPT), :],
                            acc.at[pl.ds(s * RPT, RPT), :])
            idx_base = pi * (NS * RPT_IDX) + s * RPT_IDX
            pltpu.sync_copy(sidx.at[pl.ds(idx_base, RPT_IDX), :], sidx_v)
            pltpu.sync_copy(didx.at[pl.ds(idx_base, RPT_IDX), :], didx_v)
            for b in range(NBUF):
                pltpu.async_copy(tables.at[sidx_v.at[b]], rows_v.at[b],
                                 sems[b])
            plsc.subcore_barrier()

            @pl.loop(0, N_GRP - 1)
            def _(g):
                for b in range(NBUF):
                    pltpu.make_async_copy(tables.at[pl.ds(0, CH), :],
                                          rows_v.at[b], sems[b]).wait()
                    pltpu.sync_copy(rows_v.at[b],
                                    acc.at[didx_v.at[g * NBUF + b]],
                                    add=True)
                    pltpu.async_copy(
                        tables.at[sidx_v.at[(g + 1) * NBUF + b]],
                        rows_v.at[b], sems[b])

            for b in range(NBUF):
                pltpu.make_async_copy(tables.at[pl.ds(0, CH), :],
                                      rows_v.at[b], sems[b]).wait()
                pltpu.sync_copy(rows_v.at[b],
                                acc.at[didx_v.at[(N_GRP - 1) * NBUF + b]],
                                add=True)
            plsc.subcore_barrier()
            pltpu.sync_copy(acc.at[pl.ds(s * RPT, RPT), :],
                            out.at[pl.ds(pi * N_ACC + s * RPT, RPT), :])
            plsc.subcore_barrier()

    return k


def _lrelu(v):
    return jnp.where(v >= 0, v, 0.2 * v)


def _tc1(deg4, xs4, w1a, w1b):
    def body(deg_ref, x_ref, wa_ref, wb_ref, ha_ref, hb_ref, dv_ref):
        dv = 1.0 / jnp.sqrt(deg_ref[0])
        x = x_ref[0]
        ha_ref[0] = jnp.dot(x, wa_ref[0],
                            preferred_element_type=jnp.float32) * dv
        hb_ref[0] = jnp.dot(x, wb_ref[0],
                            preferred_element_type=jnp.float32) * dv
        dv_ref[0] = dv

    return pl.pallas_call(
        body,
        grid=(4, N // BM),
        in_specs=[pl.BlockSpec((1, BM, 1), lambda i, j: (i, j, 0)),
                  pl.BlockSpec((1, BM, D_IN), lambda i, j: (i, j, 0)),
                  pl.BlockSpec((1, D_IN, DW), lambda i, j: (i, 0, 0)),
                  pl.BlockSpec((1, D_IN, DW), lambda i, j: (i, 0, 0))],
        out_specs=[pl.BlockSpec((1, BM, DW), lambda i, j: (i, j, 0)),
                   pl.BlockSpec((1, BM, DW), lambda i, j: (i, j, 0)),
                   pl.BlockSpec((1, BM, 1), lambda i, j: (i, j, 0))],
        out_shape=[jax.ShapeDtypeStruct((4, N, DW), jnp.float32),
                   jax.ShapeDtypeStruct((4, N, DW), jnp.float32),
                   jax.ShapeDtypeStruct((4, N, 1), jnp.float32)],
    )(deg4, xs4, w1a, w1b)


def _tc2(acc1a, acc1b, dinv, b1a, b1b, w2a, w2b):
    def body(aa_ref, ab_ref, dv_ref, ba_ref, bb_ref, wa_ref, wb_ref, o_ref):
        dv = dv_ref[0]
        za = _lrelu(aa_ref[0] * dv + ba_ref[0])
        zb = _lrelu(ab_ref[0] * dv + bb_ref[0])
        o = (jnp.dot(za, wa_ref[0], preferred_element_type=jnp.float32) +
             jnp.dot(zb, wb_ref[0], preferred_element_type=jnp.float32))
        o_ref[0] = o * dv

    return pl.pallas_call(
        body,
        grid=(4, N // BM),
        in_specs=[pl.BlockSpec((1, BM, DW), lambda i, j: (i, j, 0)),
                  pl.BlockSpec((1, BM, DW), lambda i, j: (i, j, 0)),
                  pl.BlockSpec((1, BM, 1), lambda i, j: (i, j, 0)),
                  pl.BlockSpec((1, 1, DW), lambda i, j: (i, 0, 0)),
                  pl.BlockSpec((1, 1, DW), lambda i, j: (i, 0, 0)),
                  pl.BlockSpec((1, DW, D_OUT), lambda i, j: (i, 0, 0)),
                  pl.BlockSpec((1, DW, D_OUT), lambda i, j: (i, 0, 0))],
        out_specs=pl.BlockSpec((1, BM, D_OUT), lambda i, j: (i, j, 0)),
        out_shape=jax.ShapeDtypeStruct((4, N, D_OUT), jnp.float32),
    )(acc1a, acc1b, dinv, b1a, b1b, w2a, w2b)


def _tc3(acc2, dinv, b2s):
    def body(a_ref, dv_ref, b_ref, o_ref):
        o_ref[0] = _lrelu(a_ref[0] * dv_ref[0] + b_ref[0])

    return pl.pallas_call(
        body,
        grid=(4, N // BM),
        in_specs=[pl.BlockSpec((1, BM, D_OUT), lambda i, j: (i, j, 0)),
                  pl.BlockSpec((1, BM, 1), lambda i, j: (i, j, 0)),
                  pl.BlockSpec((1, 1, D_OUT), lambda i, j: (i, 0, 0))],
        out_specs=pl.BlockSpec((1, BM, D_OUT), lambda i, j: (i, j, 0)),
        out_shape=jax.ShapeDtypeStruct((4, N, D_OUT), jnp.float32),
    )(acc2, dinv, b2s)


def _tc4(o4, xj2, wres, bres):
    def body(o1_ref, o2_ref, x_ref, w_ref, b_ref, c_ref):
        r = jnp.dot(x_ref[0], w_ref[0], preferred_element_type=jnp.float32)
        c_ref[0] = (o1_ref[0] + o2_ref[0]) * 0.5 + r + b_ref[0]

    return pl.pallas_call(
        body,
        grid=(2, N // BM),
        in_specs=[pl.BlockSpec((1, BM, D_OUT), lambda i, j: (i, j, 0)),
                  pl.BlockSpec((1, BM, D_OUT), lambda i, j: (i + 2, j, 0)),
                  pl.BlockSpec((1, BM, D_IN), lambda i, j: (i, j, 0)),
                  pl.BlockSpec((1, D_IN, D_OUT), lambda i, j: (i, 0, 0)),
                  pl.BlockSpec((1, 1, D_OUT), lambda i, j: (i, 0, 0))],
        out_specs=pl.BlockSpec((1, BM, D_OUT), lambda i, j: (i, j, 0)),
        out_shape=jax.ShapeDtypeStruct((2, N, D_OUT), jnp.float32),
    )(o4, o4, xj2, wres, bres)


def kernel(x_lnc_jaccard, x_prot_jaccard, x_lnc_blast, x_prot_blast,
           edge_index_lnc_jaccard, edge_index_prot_jaccard,
           edge_index_lnc_blast, edge_index_prot_blast,
           W_j1_lnc, b_j1_lnc, W_j1_prot, b_j1_prot,
           W_j2_lnc, b_j2_lnc, W_j2_prot, b_j2_prot,
           W_b1_lnc, b_b1_lnc, W_b1_prot, b_b1_prot,
           W_b2_lnc, b_b2_lnc, W_b2_prot, b_b2_prot,
           W_res_lnc, b_res_lnc, W_res_prot, b_res_prot):
    eis = (edge_index_lnc_jaccard, edge_index_prot_jaccard,
           edge_index_lnc_blast, edge_index_prot_blast)
    pad = jnp.arange(E_PAD - E, dtype=jnp.int32) % 16
    src_l, dst_l = [], []
    for ei in eis:
        src_l.append(jnp.concatenate([ei[0], pad]))
        dst_l.append(jnp.concatenate([ei[1], N + pad]))
    src4 = jnp.stack(src_l).reshape(4, NS * RPT_IDX, CH)
    dst4 = jnp.stack(dst_l).reshape(4, NS * RPT_IDX, CH)

    off4 = (jnp.arange(4, dtype=jnp.int32) * N_ACC)[:, None, None]
    sidx4 = (src4 + off4).reshape(4 * NS * RPT_IDX, CH)
    didx4 = dst4.reshape(4 * NS * RPT_IDX, CH)
    off8 = (jnp.arange(8, dtype=jnp.int32) * N_ACC)[:, None, None]
    sidx8 = (jnp.repeat(src4, 2, axis=0) + off8).reshape(8 * NS * RPT_IDX, CH)
    didx8 = jnp.repeat(dst4, 2, axis=0).reshape(8 * NS * RPT_IDX, CH)

    deg4 = _deg_kernel()(didx4).reshape(4, N_DEG, 1)[:, :N]

    xs4 = jnp.stack([x_lnc_jaccard, x_prot_jaccard, x_lnc_blast, x_prot_blast])
    w1s = jnp.stack([W_j1_lnc, W_j1_prot, W_b1_lnc, W_b1_prot])
    h1pa, h1pb, dinv = _tc1(deg4, xs4, w1s[:, :, :DW], w1s[:, :, DW:])

    t1 = jnp.stack([h1pa, h1pb], axis=1)
    t1 = jnp.pad(t1, ((0, 0), (0, 0), (0, N_ACC - N), (0, 0)))
    acc1 = _agg_kernel(4)(t1.reshape(8 * N_ACC, DW), sidx8, didx8)
    acc1 = acc1.reshape(4, 2, N_ACC, DW)[:, :, :N]

    b1s = jnp.stack([b_j1_lnc, b_j1_prot, b_b1_lnc, b_b1_prot])[:, None, :]
    w2s = jnp.stack([W_j2_lnc, W_j2_prot, W_b2_lnc, W_b2_prot])
    h2p = _tc2(acc1[:, 0], acc1[:, 1], dinv,
               b1s[:, :, :DW], b1s[:, :, DW:],
               w2s[:, :DW, :], w2s[:, DW:, :])

    t2 = jnp.pad(h2p, ((0, 0), (0, N_ACC - N), (0, 0)))
    acc2 = _agg_kernel(2)(t2.reshape(4 * N_ACC, D_OUT), sidx4, didx4)
    acc2 = acc2.reshape(4, N_ACC, D_OUT)[:, :N]

    b2s = jnp.stack([b_j2_lnc, b_j2_prot, b_b2_lnc, b_b2_prot])[:, None, :]
    o4 = _tc3(acc2, dinv, b2s)

    xj2 = jnp.stack([x_lnc_jaccard, x_prot_jaccard])
    wres = jnp.stack([W_res_lnc, W_res_prot])
    bres = jnp.stack([b_res_lnc, b_res_prot])[:, None, :]
    comb = _tc4(o4, xj2, wres, bres)

    return (comb[0], comb[1], o4[0], o4[1], o4[2], o4[3])


# speedup vs baseline: 28.5047x; 1.0744x over previous
import functools

import jax
import jax.numpy as jnp
from jax import lax
from jax.experimental import pallas as pl
from jax.experimental.pallas import tpu as pltpu
import jax.experimental.pallas.tpu_sc as plsc

N = 10000
E = 320000
D_IN, D_HID, D_OUT = 128, 128, 64
DW = 64
NC, NS = 2, 16
CH = 128
N_BLOCKS = 20
EPT = N_BLOCKS * 8 * CH
E_PAD = EPT * NS
RPT_IDX = EPT // CH
N_ACC = N + 112
RPT = N_ACC // NS
N_DEG = 10240
DPT = N_DEG // NS
BM = 2000
NBUF = 4
N_GRP = RPT_IDX // NBUF


def _mesh():
    return plsc.VectorSubcoreMesh(core_axis_name="c", subcore_axis_name="s",
                                  num_cores=NC, num_subcores=NS)


@functools.cache
def _deg_kernel():
    @functools.partial(
        pl.kernel, mesh=_mesh(),
        out_type=jax.ShapeDtypeStruct((4 * N_DEG,), jnp.float32),
        scratch_types=[
            pltpu.VMEM((8, CH), jnp.int32),
            pltpu.VMEM((DPT,), jnp.float32),
            pltpu.VMEM_SHARED((N_DEG,), jnp.float32),
        ],
        compiler_params=pltpu.CompilerParams(use_tc_tiling_on_sc=False),
        name="gcn_deg",
    )
    def k(didx, out, didx_v, ones_v, deg):
        c = lax.axis_index("c")
        s = lax.axis_index("s")
        for i in range(DPT // 16):
            ones_v[pl.ds(i * 16, 16)] = jnp.ones((16,), jnp.float32)

        @pl.loop(0, 2)
        def _prod(p):
            pi = 2 * c + p
            pltpu.sync_copy(ones_v, deg.at[pl.ds(s * DPT, DPT)])
            plsc.subcore_barrier()
            idx_base = pi * (NS * RPT_IDX) + s * RPT_IDX

            @pl.loop(0, N_BLOCKS)
            def _(b):
                pltpu.sync_copy(didx.at[pl.ds(idx_base + b * 8, 8), :],
                                didx_v)
                for j in range(8):
                    pltpu.sync_copy(ones_v.at[pl.ds(0, CH)],
                                    deg.at[didx_v.at[j]], add=True)

            plsc.subcore_barrier()
            pltpu.sync_copy(deg.at[pl.ds(s * DPT, DPT)],
                            out.at[pl.ds(pi * N_DEG + s * DPT, DPT)])
            plsc.subcore_barrier()

    return k


@functools.cache
def _agg_kernel(ppc):
    hpp = ppc // 2
    nprod = NC * ppc

    @functools.partial(
        pl.kernel, mesh=_mesh(),
        out_type=jax.ShapeDtypeStruct((nprod * N_ACC, DW), jnp.float32),
        scratch_types=[
            pltpu.VMEM((RPT_IDX, CH), jnp.int32),
            pltpu.VMEM((RPT_IDX, CH), jnp.int32),
            pltpu.VMEM((NBUF, CH, DW), jnp.float32),
            pltpu.VMEM_SHARED((N_ACC, DW), jnp.float32),
            pltpu.SemaphoreType.DMA,
            pltpu.SemaphoreType.DMA,
            pltpu.SemaphoreType.DMA,
            pltpu.SemaphoreType.DMA,
        ],
        compiler_params=pltpu.CompilerParams(use_tc_tiling_on_sc=False),
        name=f"gcn_agg_p{ppc}",
    )
    def k(tables, sidx, didx, out, sidx_v, didx_v, rows_v, acc,
          sem0, sem1, sem2, sem3):
        sems = (sem0, sem1, sem2, sem3)
        c = lax.axis_index("c")
        s = lax.axis_index("s")

        @pl.loop(0, ppc)
        def _prod(p):
            pi = ppc * c + p
            table = tables.at[pl.ds(pi * N_ACC, N_ACC), :]
            pltpu.sync_copy(table.at[pl.ds(s * RPT, RPT), :],
                            acc.at[pl.ds(s * RPT, RPT), :])
            idx_base = (pi // hpp) * (NS * RPT_IDX) + s * RPT_IDX
            pltpu.sync_copy(sidx.at[pl.ds(idx_base, RPT_IDX), :], sidx_v)
            pltpu.sync_copy(didx.at[pl.ds(idx_base, RPT_IDX), :], didx_v)
            for b in range(NBUF):
                pltpu.async_copy(table.at[sidx_v.at[b]], rows_v.at[b],
                                 sems[b])
            plsc.subcore_barrier()

            @pl.loop(0, N_GRP - 1)
            def _(g):
                for b in range(NBUF):
                    pltpu.make_async_copy(tables.at[pl.ds(0, CH), :],
                                          rows_v.at[b], sems[b]).wait()
                    pltpu.sync_copy(rows_v.at[b],
                                    acc.at[didx_v.at[g * NBUF + b]],
                                    add=True)
                    pltpu.async_copy(
                        table.at[sidx_v.at[(g + 1) * NBUF + b]],
                        rows_v.at[b], sems[b])

            for b in range(NBUF):
                pltpu.make_async_copy(tables.at[pl.ds(0, CH), :],
                                      rows_v.at[b], sems[b]).wait()
                pltpu.sync_copy(rows_v.at[b],
                                acc.at[didx_v.at[(N_GRP - 1) * NBUF + b]],
                                add=True)
            plsc.subcore_barrier()
            pltpu.sync_copy(acc.at[pl.ds(s * RPT, RPT), :],
                            out.at[pl.ds(pi * N_ACC + s * RPT, RPT), :])
            plsc.subcore_barrier()

    return k


def _lrelu(v):
    return jnp.where(v >= 0, v, 0.2 * v)


def _tc1(deg4, xs4, w1h):
    def body(deg_ref, x_ref, w_ref, hp_ref, dv_ref):
        dv = 1.0 / jnp.sqrt(deg_ref[0])
        hp_ref[0, 0] = jnp.dot(x_ref[0], w_ref[0, 0],
                               preferred_element_type=jnp.float32) * dv
        dv_ref[0] = dv

    return pl.pallas_call(
        body,
        grid=(4, N // BM, 2),
        in_specs=[pl.BlockSpec((1, BM, 1), lambda i, j, h: (i, j, 0)),
                  pl.BlockSpec((1, BM, D_IN), lambda i, j, h: (i, j, 0)),
                  pl.BlockSpec((1, 1, D_IN, DW), lambda i, j, h: (i, h, 0, 0))],
        out_specs=[pl.BlockSpec((1, 1, BM, DW), lambda i, j, h: (i, h, j, 0)),
                   pl.BlockSpec((1, BM, 1), lambda i, j, h: (i, j, 0))],
        out_shape=[jax.ShapeDtypeStruct((4, 2, N_ACC, DW), jnp.float32),
                   jax.ShapeDtypeStruct((4, N, 1), jnp.float32)],
    )(deg4, xs4, w1h)


def _tc2(acc1, dinv, b1s, w2s):
    def body(aa_ref, ab_ref, dv_ref, b_ref, wa_ref, wb_ref, o_ref):
        dv = dv_ref[0]
        za = _lrelu(aa_ref[0, 0] * dv + b_ref[0, :, :DW])
        zb = _lrelu(ab_ref[0, 0] * dv + b_ref[0, :, DW:])
        o = (jnp.dot(za, wa_ref[0], preferred_element_type=jnp.float32) +
             jnp.dot(zb, wb_ref[0], preferred_element_type=jnp.float32))
        o_ref[0] = o * dv

    return pl.pallas_call(
        body,
        grid=(4, N // BM),
        in_specs=[pl.BlockSpec((1, 1, BM, DW), lambda i, j: (i, 0, j, 0)),
                  pl.BlockSpec((1, 1, BM, DW), lambda i, j: (i, 1, j, 0)),
                  pl.BlockSpec((1, BM, 1), lambda i, j: (i, j, 0)),
                  pl.BlockSpec((1, 1, D_HID), lambda i, j: (i, 0, 0)),
                  pl.BlockSpec((1, DW, D_OUT), lambda i, j: (i, 0, 0)),
                  pl.BlockSpec((1, DW, D_OUT), lambda i, j: (i, 0, 0))],
        out_specs=pl.BlockSpec((1, BM, D_OUT), lambda i, j: (i, j, 0)),
        out_shape=jax.ShapeDtypeStruct((4, N_ACC, D_OUT), jnp.float32),
    )(acc1, acc1, dinv, b1s, w2s[:, :DW, :], w2s[:, DW:, :])


def _tc3(acc2, dinv, b2s):
    def body(a_ref, dv_ref, b_ref, o_ref):
        o_ref[0] = _lrelu(a_ref[0] * dv_ref[0] + b_ref[0])

    return pl.pallas_call(
        body,
        grid=(4, N // BM),
        in_specs=[pl.BlockSpec((1, BM, D_OUT), lambda i, j: (i, j, 0)),
                  pl.BlockSpec((1, BM, 1), lambda i, j: (i, j, 0)),
                  pl.BlockSpec((1, 1, D_OUT), lambda i, j: (i, 0, 0))],
        out_specs=pl.BlockSpec((1, BM, D_OUT), lambda i, j: (i, j, 0)),
        out_shape=jax.ShapeDtypeStruct((4, N, D_OUT), jnp.float32),
    )(acc2, dinv, b2s)


def _tc4(o4, xs4, wres, bres):
    def body(o1_ref, o2_ref, x_ref, w_ref, b_ref, c_ref):
        r = jnp.dot(x_ref[0], w_ref[0], preferred_element_type=jnp.float32)
        c_ref[0] = (o1_ref[0] + o2_ref[0]) * 0.5 + r + b_ref[0]

    return pl.pallas_call(
        body,
        grid=(2, N // BM),
        in_specs=[pl.BlockSpec((1, BM, D_OUT), lambda i, j: (i, j, 0)),
                  pl.BlockSpec((1, BM, D_OUT), lambda i, j: (i + 2, j, 0)),
                  pl.BlockSpec((1, BM, D_IN), lambda i, j: (i, j, 0)),
                  pl.BlockSpec((1, D_IN, D_OUT), lambda i, j: (i, 0, 0)),
                  pl.BlockSpec((1, 1, D_OUT), lambda i, j: (i, 0, 0))],
        out_specs=pl.BlockSpec((1, BM, D_OUT), lambda i, j: (i, j, 0)),
        out_shape=jax.ShapeDtypeStruct((2, N, D_OUT), jnp.float32),
    )(o4, o4, xs4, wres, bres)


def kernel(x_lnc_jaccard, x_prot_jaccard, x_lnc_blast, x_prot_blast,
           edge_index_lnc_jaccard, edge_index_prot_jaccard,
           edge_index_lnc_blast, edge_index_prot_blast,
           W_j1_lnc, b_j1_lnc, W_j1_prot, b_j1_prot,
           W_j2_lnc, b_j2_lnc, W_j2_prot, b_j2_prot,
           W_b1_lnc, b_b1_lnc, W_b1_prot, b_b1_prot,
           W_b2_lnc, b_b2_lnc, W_b2_prot, b_b2_prot,
           W_res_lnc, b_res_lnc, W_res_prot, b_res_prot):
    eis = (edge_index_lnc_jaccard, edge_index_prot_jaccard,
           edge_index_lnc_blast, edge_index_prot_blast)
    pad = jnp.arange(E_PAD - E, dtype=jnp.int32) % 16
    src_l, dst_l = [], []
    for ei in eis:
        src_l.append(jnp.concatenate([ei[0], pad]))
        dst_l.append(jnp.concatenate([ei[1], N + pad]))
    sidx = jnp.stack(src_l).reshape(4 * NS * RPT_IDX, CH)
    didx = jnp.stack(dst_l).reshape(4 * NS * RPT_IDX, CH)

    deg4 = _deg_kernel()(didx).reshape(4, N_DEG, 1)

    xs4 = jnp.stack([x_lnc_jaccard, x_prot_jaccard, x_lnc_blast, x_prot_blast])
    w1s = jnp.stack([W_j1_lnc, W_j1_prot, W_b1_lnc, W_b1_prot])
    w1h = jnp.stack([w1s[:, :, :DW], w1s[:, :, DW:]], axis=1)
    h1p, dinv = _tc1(deg4, xs4, w1h)

    acc1 = _agg_kernel(4)(h1p.reshape(8 * N_ACC, DW), sidx, didx)
    acc1 = acc1.reshape(4, 2, N_ACC, DW)

    b1s = jnp.stack([b_j1_lnc, b_j1_prot, b_b1_lnc, b_b1_prot])[:, None, :]
    w2s = jnp.stack([W_j2_lnc, W_j2_prot, W_b2_lnc, W_b2_prot])
    h2p = _tc2(acc1, dinv, b1s, w2s)

    acc2 = _agg_kernel(2)(h2p.reshape(4 * N_ACC, D_OUT), sidx, didx)
    acc2 = acc2.reshape(4, N_ACC, D_OUT)[:, :N]

    b2s = jnp.stack([b_j2_lnc, b_j2_prot, b_b2_lnc, b_b2_prot])[:, None, :]
    o4 = _tc3(acc2, dinv, b2s)

    wres = jnp.stack([W_res_lnc, W_res_prot])
    bres = jnp.stack([b_res_lnc, b_res_prot])[:, None, :]
    comb = _tc4(o4, xs4[:2], wres, bres)

    return (comb[0], comb[1], o4[0], o4[1], o4[2], o4[3])
